# baseline (speedup 1.0000x reference)
import functools

import jax
import jax.numpy as jnp
from jax import lax
from jax.experimental import pallas as pl
from jax.experimental.pallas import tpu as pltpu
from jax.experimental.pallas import tpu_sc as plsc

_BATCH = 16384
_EMB_K = 32


def _build():
    info = plsc.get_sparse_core_info()
    nc, ns, lanes = info.num_cores, info.num_subcores, info.num_lanes
    nw = nc * ns
    b_per_w = _BATCH // nw
    n_chunks = b_per_w // 128
    groups = b_per_w // lanes
    gpc = 128 // lanes

    mesh = plsc.VectorSubcoreMesh(core_axis_name="c", subcore_axis_name="s")

    @functools.partial(
        pl.kernel,
        out_type=jax.ShapeDtypeStruct((_BATCH,), jnp.float32),
        mesh=mesh,
        compiler_params=pltpu.CompilerParams(
            needs_layout_passes=False, use_tc_tiling_on_sc=False),
        scratch_types=[
            pltpu.VMEM((n_chunks, 128), jnp.int32),
            pltpu.VMEM((n_chunks, 128), jnp.int32),
            *[pltpu.VMEM((128, _EMB_K), jnp.float32)
              for _ in range(n_chunks)],
            *[pltpu.VMEM((128, _EMB_K), jnp.float32)
              for _ in range(n_chunks)],
            pltpu.VMEM((2 * _EMB_K,), jnp.float32),
            pltpu.VMEM((16,), jnp.float32),
            pltpu.VMEM((b_per_w,), jnp.float32),
            pltpu.SemaphoreType.DMA,
        ],
    )
    def ncf(uid_hbm, vid_hbm, w_hbm, h_hbm, lw_hbm, b_hbm, out_hbm,
            idx_u, idx_v, *rest):
        rows_u = rest[:n_chunks]
        rows_v = rest[n_chunks:2 * n_chunks]
        wvm, bvm, out_v, sem = rest[2 * n_chunks:]
        wid = lax.axis_index("s") * nc + lax.axis_index("c")
        row0 = wid * n_chunks
        pltpu.sync_copy(uid_hbm.at[pl.ds(row0, n_chunks)], idx_u)
        pltpu.sync_copy(vid_hbm.at[pl.ds(row0, n_chunks)], idx_v)
        pltpu.sync_copy(lw_hbm, wvm)
        pltpu.sync_copy(b_hbm, bvm)

        copies = []
        for c in range(n_chunks):
            copies.append(pltpu.async_copy(w_hbm.at[idx_u.at[c]],
                                           rows_u[c], sem))
            copies.append(pltpu.async_copy(h_hbm.at[idx_v.at[c]],
                                           rows_v[c], sem))
        for cp in copies:
            cp.wait()

        iota = lax.iota(jnp.int32, lanes)
        wregs = [wvm[pl.ds(j * lanes, lanes)] for j in range(2 * _EMB_K // lanes)]
        wk = [wregs[k // lanes][k % lanes] for k in range(2 * _EMB_K)]
        kcols = [jnp.full((lanes,), k, jnp.int32) for k in range(_EMB_K)]

        for c in range(n_chunks):
            ru, rv = rows_u[c], rows_v[c]

            def body(gi, carry, ru=ru, rv=rv, c=c):
                rows = gi * lanes + iota
                acc = bvm[:]
                for k in range(_EMB_K):
                    acc = acc + plsc.load_gather(ru, [rows, kcols[k]]) * wk[k]
                for k in range(_EMB_K):
                    acc = acc + (plsc.load_gather(rv, [rows, kcols[k]])
                                 * wk[_EMB_K + k])
                out_v[pl.ds((c * gpc + gi) * lanes, lanes)] = (
                    1.0 / (1.0 + jnp.exp(-acc)))
                return carry

            lax.fori_loop(0, gpc, body, 0)
        pltpu.sync_copy(out_v, out_hbm.at[pl.ds(wid * b_per_w, b_per_w)])

    return ncf


_NCF = _build()


def kernel(x, W, H, lin_w, lin_b):
    uid = x[:, 0].astype(jnp.int32).reshape(128, 128)
    vid = x[:, 1].astype(jnp.int32).reshape(128, 128)
    lw = lin_w.reshape(2 * _EMB_K).astype(jnp.float32)
    bb = jnp.broadcast_to(lin_b.astype(jnp.float32).reshape(1), (16,))
    return _NCF(uid, vid, W, H, lw, bb)


# speedup vs baseline: 1.6342x; 1.6342x over previous
import functools

import jax
import jax.numpy as jnp
from jax import lax
from jax.experimental import pallas as pl
from jax.experimental.pallas import tpu as pltpu
from jax.experimental.pallas import tpu_sc as plsc

_BATCH = 16384
_EMB_K = 32
_ROWS = 1000000
_BU = 1024
_NBLK = (_ROWS + _BU - 1) // _BU
_SROWS = _NBLK * _BU // 128


def _tc_fold(wt_ref, ht_ref, wu_ref, wv_ref, sw_ref, sh_ref):
    wu = wu_ref[...]
    wv = wv_ref[...]
    wb = wt_ref[...]
    hb = ht_ref[...]
    for j in range(_BU // 128):
        sl = slice(j * 128, (j + 1) * 128)
        sw_ref[j, :] = jnp.sum(wb[:, sl] * wu, axis=0)
        sh_ref[j, :] = jnp.sum(hb[:, sl] * wv, axis=0)


def _fold_tables(wt, ht, wu, wv):
    grid = (_NBLK,)
    out_shape = [
        jax.ShapeDtypeStruct((_SROWS, 128), jnp.float32),
        jax.ShapeDtypeStruct((_SROWS, 128), jnp.float32),
    ]
    sw, sh = pl.pallas_call(
        _tc_fold,
        grid=grid,
        in_specs=[
            pl.BlockSpec((_EMB_K, _BU), lambda i: (0, i)),
            pl.BlockSpec((_EMB_K, _BU), lambda i: (0, i)),
            pl.BlockSpec((_EMB_K, 128), lambda i: (0, 0)),
            pl.BlockSpec((_EMB_K, 128), lambda i: (0, 0)),
        ],
        out_specs=[
            pl.BlockSpec((_BU // 128, 128), lambda i: (i, 0)),
            pl.BlockSpec((_BU // 128, 128), lambda i: (i, 0)),
        ],
        out_shape=out_shape,
        compiler_params=pltpu.CompilerParams(
            dimension_semantics=("arbitrary",)),
    )(wt, ht, wu, wv)
    return sw.reshape(_SROWS * 128), sh.reshape(_SROWS * 128)


def _build_sc():
    info = plsc.get_sparse_core_info()
    nc, ns, lanes = info.num_cores, info.num_subcores, info.num_lanes
    nw = nc * ns
    b_per_w = _BATCH // nw
    n_chunks = b_per_w // 128

    mesh = plsc.VectorSubcoreMesh(core_axis_name="c", subcore_axis_name="s")

    @functools.partial(
        pl.kernel,
        out_type=jax.ShapeDtypeStruct((_BATCH,), jnp.float32),
        mesh=mesh,
        compiler_params=pltpu.CompilerParams(
            needs_layout_passes=False, use_tc_tiling_on_sc=False),
        scratch_types=[
            pltpu.VMEM((n_chunks, 128), jnp.int32),
            pltpu.VMEM((n_chunks, 128), jnp.int32),
            pltpu.VMEM((n_chunks, 128), jnp.float32),
            pltpu.VMEM((n_chunks, 128), jnp.float32),
            pltpu.VMEM((16,), jnp.float32),
            pltpu.VMEM((b_per_w,), jnp.float32),
            pltpu.SemaphoreType.DMA,
        ],
    )
    def ncf(uid_hbm, vid_hbm, sw_hbm, sh_hbm, b_hbm, out_hbm,
            idx_u, idx_v, zu, zv, bvm, out_v, sem):
        wid = lax.axis_index("s") * nc + lax.axis_index("c")
        row0 = wid * n_chunks
        pltpu.sync_copy(uid_hbm.at[pl.ds(row0, n_chunks)], idx_u)
        pltpu.sync_copy(vid_hbm.at[pl.ds(row0, n_chunks)], idx_v)
        pltpu.sync_copy(b_hbm, bvm)

        copies = []
        for c in range(n_chunks):
            copies.append(pltpu.async_copy(sw_hbm.at[idx_u.at[c]],
                                           zu.at[c], sem))
            copies.append(pltpu.async_copy(sh_hbm.at[idx_v.at[c]],
                                           zv.at[c], sem))
        for cp in copies:
            cp.wait()

        bias = bvm[:]
        for c in range(n_chunks):
            for j in range(128 // lanes):
                z = (zu[c, pl.ds(j * lanes, lanes)]
                     + zv[c, pl.ds(j * lanes, lanes)] + bias)
                out_v[pl.ds((c * 128 + j * lanes), lanes)] = (
                    1.0 / (1.0 + jnp.exp(-z)))

        pltpu.sync_copy(out_v, out_hbm.at[pl.ds(wid * b_per_w, b_per_w)])

    return ncf


_NCF_SC = _build_sc()


def kernel(x, W, H, lin_w, lin_b):
    uid = x[:, 0].astype(jnp.int32).reshape(128, 128)
    vid = x[:, 1].astype(jnp.int32).reshape(128, 128)
    lw = lin_w.reshape(2 * _EMB_K).astype(jnp.float32)
    wu = jnp.broadcast_to(lw[:_EMB_K, None], (_EMB_K, 128))
    wv = jnp.broadcast_to(lw[_EMB_K:, None], (_EMB_K, 128))
    sw, sh = _fold_tables(W.T, H.T, wu, wv)
    bb = jnp.broadcast_to(lin_b.astype(jnp.float32).reshape(1), (16,))
    return _NCF_SC(uid, vid, sw, sh, bb)


# speedup vs baseline: 6.1711x; 3.7761x over previous
import functools

import jax
import jax.numpy as jnp
from jax import lax
from jax.experimental import pallas as pl
from jax.experimental.pallas import tpu as pltpu
from jax.experimental.pallas import tpu_sc as plsc

_BATCH = 16384
_EMB_K = 32
_ROWS = 1000000
_BU = 8192
_NBLK = (_ROWS + _BU - 1) // _BU
_SROWS = _NBLK * _BU // 128


def _tc_fold(wt_ref, ht_ref, wu_ref, wv_ref, sw_ref, sh_ref):
    wu = wu_ref[...]
    wv = wv_ref[...]
    wb = wt_ref[...]
    hb = ht_ref[...]
    for j in range(_BU // 128):
        sl = slice(j * 128, (j + 1) * 128)
        sw_ref[j, :] = jnp.sum(wb[:, sl] * wu, axis=0)
        sh_ref[j, :] = jnp.sum(hb[:, sl] * wv, axis=0)


def _fold_tables(wt, ht, wu, wv):
    grid = (_NBLK,)
    out_shape = [
        jax.ShapeDtypeStruct((_SROWS, 128), jnp.float32),
        jax.ShapeDtypeStruct((_SROWS, 128), jnp.float32),
    ]
    sw, sh = pl.pallas_call(
        _tc_fold,
        grid=grid,
        in_specs=[
            pl.BlockSpec((_EMB_K, _BU), lambda i: (0, i)),
            pl.BlockSpec((_EMB_K, _BU), lambda i: (0, i)),
            pl.BlockSpec((_EMB_K, 128), lambda i: (0, 0)),
            pl.BlockSpec((_EMB_K, 128), lambda i: (0, 0)),
        ],
        out_specs=[
            pl.BlockSpec((_BU // 128, 128), lambda i: (i, 0)),
            pl.BlockSpec((_BU // 128, 128), lambda i: (i, 0)),
        ],
        out_shape=out_shape,
        compiler_params=pltpu.CompilerParams(
            dimension_semantics=("arbitrary",)),
    )(wt, ht, wu, wv)
    return sw.reshape(_SROWS * 128), sh.reshape(_SROWS * 128)


def _build_sc():
    info = plsc.get_sparse_core_info()
    nc, ns, lanes = info.num_cores, info.num_subcores, info.num_lanes
    nw = nc * ns
    b_per_w = _BATCH // nw
    n_chunks = b_per_w // 128

    mesh = plsc.VectorSubcoreMesh(core_axis_name="c", subcore_axis_name="s")

    @functools.partial(
        pl.kernel,
        out_type=jax.ShapeDtypeStruct((_BATCH,), jnp.float32),
        mesh=mesh,
        compiler_params=pltpu.CompilerParams(
            needs_layout_passes=False, use_tc_tiling_on_sc=False),
        scratch_types=[
            pltpu.VMEM((n_chunks, 128), jnp.int32),
            pltpu.VMEM((n_chunks, 128), jnp.int32),
            pltpu.VMEM((n_chunks, 128), jnp.float32),
            pltpu.VMEM((n_chunks, 128), jnp.float32),
            pltpu.VMEM((16,), jnp.float32),
            pltpu.VMEM((b_per_w,), jnp.float32),
            pltpu.SemaphoreType.DMA,
        ],
    )
    def ncf(uid_hbm, vid_hbm, sw_hbm, sh_hbm, b_hbm, out_hbm,
            idx_u, idx_v, zu, zv, bvm, out_v, sem):
        wid = lax.axis_index("s") * nc + lax.axis_index("c")
        row0 = wid * n_chunks
        pltpu.sync_copy(uid_hbm.at[pl.ds(row0, n_chunks)], idx_u)
        pltpu.sync_copy(vid_hbm.at[pl.ds(row0, n_chunks)], idx_v)
        pltpu.sync_copy(b_hbm, bvm)

        copies = []
        for c in range(n_chunks):
            copies.append(pltpu.async_copy(sw_hbm.at[idx_u.at[c]],
                                           zu.at[c], sem))
            copies.append(pltpu.async_copy(sh_hbm.at[idx_v.at[c]],
                                           zv.at[c], sem))
        for cp in copies:
            cp.wait()

        bias = bvm[:]
        for c in range(n_chunks):
            for j in range(128 // lanes):
                z = (zu[c, pl.ds(j * lanes, lanes)]
                     + zv[c, pl.ds(j * lanes, lanes)] + bias)
                out_v[pl.ds((c * 128 + j * lanes), lanes)] = (
                    1.0 / (1.0 + jnp.exp(-z)))

        pltpu.sync_copy(out_v, out_hbm.at[pl.ds(wid * b_per_w, b_per_w)])

    return ncf


_NCF_SC = _build_sc()


def kernel(x, W, H, lin_w, lin_b):
    uid = x[:, 0].astype(jnp.int32).reshape(128, 128)
    vid = x[:, 1].astype(jnp.int32).reshape(128, 128)
    lw = lin_w.reshape(2 * _EMB_K).astype(jnp.float32)
    wu = jnp.broadcast_to(lw[:_EMB_K, None], (_EMB_K, 128))
    wv = jnp.broadcast_to(lw[_EMB_K:, None], (_EMB_K, 128))
    sw, sh = _fold_tables(W.T, H.T, wu, wv)
    bb = jnp.broadcast_to(lin_b.astype(jnp.float32).reshape(1), (16,))
    return _NCF_SC(uid, vid, sw, sh, bb)


# speedup vs baseline: 7.8626x; 1.2741x over previous
import functools

import jax
import jax.numpy as jnp
from jax import lax
from jax.experimental import pallas as pl
from jax.experimental.pallas import tpu as pltpu
from jax.experimental.pallas import tpu_sc as plsc

_BATCH = 16384
_EMB_K = 32
_ROWS = 1000000
_BU = 16384
_NBLK = (_ROWS + _BU - 1) // _BU
_SROWS = _NBLK * _BU // 128


def _tc_fold(wt_ref, ht_ref, wu_ref, wv_ref, sw_ref, sh_ref):
    wu = wu_ref[...]
    wv = wv_ref[...]
    wb = wt_ref[...]
    hb = ht_ref[...]
    for j in range(_BU // 128):
        sl = slice(j * 128, (j + 1) * 128)
        sw_ref[j, :] = jnp.sum(wb[:, sl] * wu, axis=0)
        sh_ref[j, :] = jnp.sum(hb[:, sl] * wv, axis=0)


def _fold_tables(wt, ht, wu, wv):
    grid = (_NBLK,)
    out_shape = [
        jax.ShapeDtypeStruct((_SROWS, 128), jnp.float32),
        jax.ShapeDtypeStruct((_SROWS, 128), jnp.float32),
    ]
    sw, sh = pl.pallas_call(
        _tc_fold,
        grid=grid,
        in_specs=[
            pl.BlockSpec((_EMB_K, _BU), lambda i: (0, i)),
            pl.BlockSpec((_EMB_K, _BU), lambda i: (0, i)),
            pl.BlockSpec((_EMB_K, 128), lambda i: (0, 0)),
            pl.BlockSpec((_EMB_K, 128), lambda i: (0, 0)),
        ],
        out_specs=[
            pl.BlockSpec((_BU // 128, 128), lambda i: (i, 0)),
            pl.BlockSpec((_BU // 128, 128), lambda i: (i, 0)),
        ],
        out_shape=out_shape,
        compiler_params=pltpu.CompilerParams(
            dimension_semantics=("arbitrary",)),
    )(wt, ht, wu, wv)
    return sw.reshape(_SROWS * 128), sh.reshape(_SROWS * 128)


def _build_sc():
    info = plsc.get_sparse_core_info()
    nc, ns, lanes = info.num_cores, info.num_subcores, info.num_lanes
    nw = nc * ns
    b_per_w = _BATCH // nw
    n_chunks = b_per_w // 128

    mesh = plsc.VectorSubcoreMesh(core_axis_name="c", subcore_axis_name="s")

    @functools.partial(
        pl.kernel,
        out_type=jax.ShapeDtypeStruct((_BATCH,), jnp.float32),
        mesh=mesh,
        compiler_params=pltpu.CompilerParams(
            needs_layout_passes=False, use_tc_tiling_on_sc=False),
        scratch_types=[
            pltpu.VMEM((n_chunks, 128), jnp.int32),
            pltpu.VMEM((n_chunks, 128), jnp.int32),
            pltpu.VMEM((n_chunks, 128), jnp.float32),
            pltpu.VMEM((n_chunks, 128), jnp.float32),
            pltpu.VMEM((16,), jnp.float32),
            pltpu.VMEM((b_per_w,), jnp.float32),
            pltpu.SemaphoreType.DMA,
        ],
    )
    def ncf(uid_hbm, vid_hbm, sw_hbm, sh_hbm, b_hbm, out_hbm,
            idx_u, idx_v, zu, zv, bvm, out_v, sem):
        wid = lax.axis_index("s") * nc + lax.axis_index("c")
        row0 = wid * n_chunks
        pltpu.sync_copy(uid_hbm.at[pl.ds(row0, n_chunks)], idx_u)
        pltpu.sync_copy(vid_hbm.at[pl.ds(row0, n_chunks)], idx_v)
        pltpu.sync_copy(b_hbm, bvm)

        copies = []
        for c in range(n_chunks):
            copies.append(pltpu.async_copy(sw_hbm.at[idx_u.at[c]],
                                           zu.at[c], sem))
            copies.append(pltpu.async_copy(sh_hbm.at[idx_v.at[c]],
                                           zv.at[c], sem))
        for cp in copies:
            cp.wait()

        bias = bvm[:]
        for c in range(n_chunks):
            for j in range(128 // lanes):
                z = (zu[c, pl.ds(j * lanes, lanes)]
                     + zv[c, pl.ds(j * lanes, lanes)] + bias)
                out_v[pl.ds((c * 128 + j * lanes), lanes)] = (
                    1.0 / (1.0 + jnp.exp(-z)))

        pltpu.sync_copy(out_v, out_hbm.at[pl.ds(wid * b_per_w, b_per_w)])

    return ncf


_NCF_SC = _build_sc()


def kernel(x, W, H, lin_w, lin_b):
    uid = x[:, 0].astype(jnp.int32).reshape(128, 128)
    vid = x[:, 1].astype(jnp.int32).reshape(128, 128)
    lw = lin_w.reshape(2 * _EMB_K).astype(jnp.float32)
    wu = jnp.broadcast_to(lw[:_EMB_K, None], (_EMB_K, 128))
    wv = jnp.broadcast_to(lw[_EMB_K:, None], (_EMB_K, 128))
    sw, sh = _fold_tables(W.T, H.T, wu, wv)
    bb = jnp.broadcast_to(lin_b.astype(jnp.float32).reshape(1), (16,))
    return _NCF_SC(uid, vid, sw, sh, bb)


# speedup vs baseline: 8.7194x; 1.1090x over previous
import functools

import jax
import jax.numpy as jnp
from jax import lax
from jax.experimental import pallas as pl
from jax.experimental.pallas import tpu as pltpu
from jax.experimental.pallas import tpu_sc as plsc

_BATCH = 16384
_EMB_K = 32
_ROWS = 1000000
_BU = 32768
_NBLK = (_ROWS + _BU - 1) // _BU
_SROWS = _NBLK * _BU // 128


def _tc_fold(wt_ref, ht_ref, wu_ref, wv_ref, sw_ref, sh_ref):
    wu = wu_ref[...]
    wv = wv_ref[...]
    wb = wt_ref[...]
    hb = ht_ref[...]
    for j in range(_BU // 128):
        sl = slice(j * 128, (j + 1) * 128)
        sw_ref[j, :] = jnp.sum(wb[:, sl] * wu, axis=0)
        sh_ref[j, :] = jnp.sum(hb[:, sl] * wv, axis=0)


def _fold_tables(wt, ht, wu, wv):
    grid = (_NBLK,)
    out_shape = [
        jax.ShapeDtypeStruct((_SROWS, 128), jnp.float32),
        jax.ShapeDtypeStruct((_SROWS, 128), jnp.float32),
    ]
    sw, sh = pl.pallas_call(
        _tc_fold,
        grid=grid,
        in_specs=[
            pl.BlockSpec((_EMB_K, _BU), lambda i: (0, i)),
            pl.BlockSpec((_EMB_K, _BU), lambda i: (0, i)),
            pl.BlockSpec((_EMB_K, 128), lambda i: (0, 0)),
            pl.BlockSpec((_EMB_K, 128), lambda i: (0, 0)),
        ],
        out_specs=[
            pl.BlockSpec((_BU // 128, 128), lambda i: (i, 0)),
            pl.BlockSpec((_BU // 128, 128), lambda i: (i, 0)),
        ],
        out_shape=out_shape,
        compiler_params=pltpu.CompilerParams(
            dimension_semantics=("arbitrary",)),
    )(wt, ht, wu, wv)
    return sw.reshape(_SROWS * 128), sh.reshape(_SROWS * 128)


def _build_sc():
    info = plsc.get_sparse_core_info()
    nc, ns, lanes = info.num_cores, info.num_subcores, info.num_lanes
    nw = nc * ns
    b_per_w = _BATCH // nw
    n_chunks = b_per_w // 128

    mesh = plsc.VectorSubcoreMesh(core_axis_name="c", subcore_axis_name="s")

    @functools.partial(
        pl.kernel,
        out_type=jax.ShapeDtypeStruct((_BATCH,), jnp.float32),
        mesh=mesh,
        compiler_params=pltpu.CompilerParams(
            needs_layout_passes=False, use_tc_tiling_on_sc=False),
        scratch_types=[
            pltpu.VMEM((n_chunks, 128), jnp.int32),
            pltpu.VMEM((n_chunks, 128), jnp.int32),
            pltpu.VMEM((n_chunks, 128), jnp.float32),
            pltpu.VMEM((n_chunks, 128), jnp.float32),
            pltpu.VMEM((16,), jnp.float32),
            pltpu.VMEM((b_per_w,), jnp.float32),
            pltpu.SemaphoreType.DMA,
        ],
    )
    def ncf(uid_hbm, vid_hbm, sw_hbm, sh_hbm, b_hbm, out_hbm,
            idx_u, idx_v, zu, zv, bvm, out_v, sem):
        wid = lax.axis_index("s") * nc + lax.axis_index("c")
        row0 = wid * n_chunks
        pltpu.sync_copy(uid_hbm.at[pl.ds(row0, n_chunks)], idx_u)
        pltpu.sync_copy(vid_hbm.at[pl.ds(row0, n_chunks)], idx_v)
        pltpu.sync_copy(b_hbm, bvm)

        copies = []
        for c in range(n_chunks):
            copies.append(pltpu.async_copy(sw_hbm.at[idx_u.at[c]],
                                           zu.at[c], sem))
            copies.append(pltpu.async_copy(sh_hbm.at[idx_v.at[c]],
                                           zv.at[c], sem))
        for cp in copies:
            cp.wait()

        bias = bvm[:]
        for c in range(n_chunks):
            for j in range(128 // lanes):
                z = (zu[c, pl.ds(j * lanes, lanes)]
                     + zv[c, pl.ds(j * lanes, lanes)] + bias)
                out_v[pl.ds((c * 128 + j * lanes), lanes)] = (
                    1.0 / (1.0 + jnp.exp(-z)))

        pltpu.sync_copy(out_v, out_hbm.at[pl.ds(wid * b_per_w, b_per_w)])

    return ncf


_NCF_SC = _build_sc()


def kernel(x, W, H, lin_w, lin_b):
    uid = x[:, 0].astype(jnp.int32).reshape(128, 128)
    vid = x[:, 1].astype(jnp.int32).reshape(128, 128)
    lw = lin_w.reshape(2 * _EMB_K).astype(jnp.float32)
    wu = jnp.broadcast_to(lw[:_EMB_K, None], (_EMB_K, 128))
    wv = jnp.broadcast_to(lw[_EMB_K:, None], (_EMB_K, 128))
    sw, sh = _fold_tables(W.T, H.T, wu, wv)
    bb = jnp.broadcast_to(lin_b.astype(jnp.float32).reshape(1), (16,))
    return _NCF_SC(uid, vid, sw, sh, bb)
